# baseline (speedup 1.0000x reference)
import functools

import jax
import jax.numpy as jnp
from jax import lax
from jax.experimental import pallas as pl
from jax.experimental.pallas import tpu as pltpu
from jax.experimental.pallas import tpu_sc as plsc

_INV_T = 20.0
_TILE_N = 1024

_N_TOTAL = 100000
_SC_ROWS = 672
_N_TC = _N_TOTAL - _SC_ROWS
assert _N_TC % _TILE_N == 0

_NC = 2
_NS = 16
_NW = _NC * _NS
_CHUNK = 16
_NCHUNKS = _SC_ROWS // _CHUNK
_RBLK = 4
_KUNROLL = 16

_K = 2048
_B = 32


def _mm_kernel(x_ref, m_ref, o_ref):
    o_ref[...] = jax.lax.dot_general(
        x_ref[...],
        m_ref[...],
        (((1,), (1,)), ((), ())),
        preferred_element_type=jnp.float32,
    ) * _INV_T


def _tc_part(x, memory):
    grid = (_N_TC // _TILE_N,)
    return pl.pallas_call(
        _mm_kernel,
        grid=grid,
        in_specs=[
            pl.BlockSpec((_B, _K), lambda i: (0, 0)),
            pl.BlockSpec((_TILE_N, _K), lambda i: (i, 0)),
        ],
        out_specs=pl.BlockSpec((_B, _TILE_N), lambda i: (0, i)),
        out_shape=jax.ShapeDtypeStruct((_B, _N_TOTAL), jnp.float32),
        compiler_params=pltpu.CompilerParams(
            dimension_semantics=("parallel",),
        ),
    )(x, memory)


def _sc_body(xf_hbm, mem_hbm, out_hbm, xf_v, m_v, o_v):
    c = lax.axis_index("c")
    s = lax.axis_index("s")
    wid = s * _NC + c

    pltpu.sync_copy(xf_hbm, xf_v)

    cnt = (_NCHUNKS - wid + _NW - 1) // _NW

    def chunk_body(t, carry):
        ci = wid + t * _NW
        r0 = _N_TC + ci * _CHUNK
        pltpu.sync_copy(mem_hbm.at[pl.ds(r0, _CHUNK)], m_v)
        for rb in range(0, _CHUNK, _RBLK):
            def k_body(kc, accs):
                new = list(accs)
                k0 = kc * _KUNROLL
                mvec = [m_v[rb + j, pl.ds(k0, _KUNROLL)] for j in range(_RBLK)]
                for u in range(_KUNROLL):
                    row = kc * 4 + u // 4
                    lane = (u % 4) * _B
                    xa = xf_v[row, pl.ds(lane, 16)]
                    xb = xf_v[row, pl.ds(lane + 16, 16)]
                    for j in range(_RBLK):
                        mm = mvec[j][u]
                        new[2 * j] = new[2 * j] + xa * mm
                        new[2 * j + 1] = new[2 * j + 1] + xb * mm
                return tuple(new)

            zero = jnp.zeros((16,), jnp.float32)
            accs = lax.fori_loop(
                0, _K // _KUNROLL, k_body, (zero,) * (2 * _RBLK)
            )
            for j in range(_RBLK):
                r = rb + j
                o_v[r // 4, pl.ds((r % 4) * _B, 16)] = accs[2 * j] * _INV_T
                o_v[r // 4, pl.ds((r % 4) * _B + 16, 16)] = (
                    accs[2 * j + 1] * _INV_T
                )
        pltpu.sync_copy(o_v, out_hbm.at[pl.ds(ci * 4, 4)])
        return carry

    lax.fori_loop(0, cnt, chunk_body, 0)


def _sc_part(xf, memory):
    mesh = plsc.VectorSubcoreMesh(core_axis_name="c", subcore_axis_name="s")
    kern = functools.partial(
        pl.kernel,
        mesh=mesh,
        out_type=jax.ShapeDtypeStruct((_SC_ROWS * _B // 128, 128), jnp.float32),
        scratch_types=[
            pltpu.VMEM((_K * _B // 128, 128), jnp.float32),
            pltpu.VMEM((_CHUNK, _K), jnp.float32),
            pltpu.VMEM((_CHUNK * _B // 128, 128), jnp.float32),
        ],
    )(_sc_body)
    return kern(xf, memory)


def kernel(x, memory):
    xf = jnp.transpose(x).reshape(_K * _B // 128, 128)
    tc_out = _tc_part(x, memory)
    sc_flat = _sc_part(xf, memory)
    sc_out = jnp.transpose(sc_flat.reshape(_SC_ROWS, _B))
    return lax.dynamic_update_slice(tc_out, sc_out, (0, _N_TC))
